# Initial kernel scaffold; baseline (speedup 1.0000x reference)
import jax
import jax.numpy as jnp
from jax.experimental import pallas as pl


def kernel(z, codebook):
    raise NotImplementedError("write your pallas kernel here")


# speedup vs baseline: 1.3294x; 1.3294x over previous
import jax
import jax.numpy as jnp
from jax.experimental import pallas as pl

_B, _T, _D, _K = 16, 1024, 64, 1024
_TT = 256
_BETA = 0.25


def _vq_body(z_ref, cb_ref, nz_ref, ne_ref, zq_ref, sim_ref, ids_ref, loss_ref):
    i = pl.program_id(0)
    j = pl.program_id(1)
    z = z_ref[...]
    cb = cb_ref[...]
    nz = nz_ref[...]
    ne = ne_ref[...]
    dot = jax.lax.dot_general(z, cb, (((1,), (1,)), ((), ())),
                              preferred_element_type=jnp.float32)
    dist = -2.0 * dot + nz + ne
    m = jnp.min(dist, axis=1, keepdims=True)
    lane = jax.lax.broadcasted_iota(jnp.int32, (_TT, _K), 1)
    idx = jnp.min(jnp.where(dist == m, lane, _K), axis=1, keepdims=True)
    oh = (lane == idx).astype(jnp.float32)
    zq = jax.lax.dot_general(oh, cb, (((1,), (0,)), ((), ())),
                             precision=jax.lax.Precision.HIGHEST,
                             preferred_element_type=jnp.float32)
    zq_ref[...] = zq
    sim_ref[...] = dot / jnp.sqrt(nz) / jnp.sqrt(ne)
    ids_ref[...] = idx
    diff = z - zq
    norms = jnp.sqrt(jnp.sum(diff * diff, axis=1, keepdims=True))
    part = jnp.sum(norms, axis=0, keepdims=True)

    @pl.when(jnp.logical_and(i == 0, j == 0))
    def _():
        loss_ref[...] = jnp.zeros_like(loss_ref)

    loss_ref[...] += part


def kernel(z, codebook):
    nz3 = jnp.sum(jnp.square(z), axis=2, keepdims=True)
    ne2 = jnp.sum(jnp.square(codebook), axis=1).reshape(1, _K)
    grid = (_B, _T // _TT)
    zq, sim, ids3, loss = pl.pallas_call(
        _vq_body,
        grid=grid,
        in_specs=[
            pl.BlockSpec((None, _TT, _D), lambda i, j: (i, j, 0)),
            pl.BlockSpec((_K, _D), lambda i, j: (0, 0)),
            pl.BlockSpec((None, _TT, 1), lambda i, j: (i, j, 0)),
            pl.BlockSpec((1, _K), lambda i, j: (0, 0)),
        ],
        out_specs=[
            pl.BlockSpec((None, _TT, _D), lambda i, j: (i, j, 0)),
            pl.BlockSpec((None, _TT, _K), lambda i, j: (i, j, 0)),
            pl.BlockSpec((None, _TT, 1), lambda i, j: (i, j, 0)),
            pl.BlockSpec((1, 1), lambda i, j: (0, 0)),
        ],
        out_shape=[
            jax.ShapeDtypeStruct((_B, _T, _D), jnp.float32),
            jax.ShapeDtypeStruct((_B, _T, _K), jnp.float32),
            jax.ShapeDtypeStruct((_B, _T, 1), jnp.int32),
            jax.ShapeDtypeStruct((1, 1), jnp.float32),
        ],
    )(z, codebook, nz3, ne2)
    ids = ids3.reshape(_B, _T)
    loss_vq = loss[0, 0] * (1.0 + _BETA) / (_B * _T)
    return zq, sim, ids, loss_vq


# speedup vs baseline: 1.8841x; 1.4173x over previous
import jax
import jax.numpy as jnp
from jax.experimental import pallas as pl

_B, _T, _D, _K = 16, 1024, 64, 1024
_TT = 256
_BETA = 0.25


def _vq_body(z_ref, cb_ref, nz_ref, ne_ref, zq_ref, sim_ref, ids_ref, loss_ref):
    i = pl.program_id(0)
    j = pl.program_id(1)
    z = z_ref[...]
    cb = cb_ref[...]
    nz = nz_ref[...]
    ne = ne_ref[...]
    dot = jax.lax.dot_general(z, cb, (((1,), (1,)), ((), ())),
                              preferred_element_type=jnp.float32)
    dist = -2.0 * dot + nz + ne
    m = jnp.min(dist, axis=1, keepdims=True)
    lane = jax.lax.broadcasted_iota(jnp.int32, (_TT, _K), 1)
    idx = jnp.min(jnp.where(dist == m, lane, _K), axis=1, keepdims=True)
    oh = (lane == idx).astype(jnp.float32)
    zq = jax.lax.dot_general(oh, cb, (((1,), (0,)), ((), ())),
                             preferred_element_type=jnp.float32)
    zq_ref[...] = zq
    sim_ref[...] = dot * jax.lax.rsqrt(nz) * jax.lax.rsqrt(ne)
    ids_ref[...] = idx
    norms = jnp.sqrt(jnp.maximum(m, 0.0))
    part = jnp.sum(norms, axis=0, keepdims=True)

    @pl.when(jnp.logical_and(i == 0, j == 0))
    def _():
        loss_ref[...] = jnp.zeros_like(loss_ref)

    loss_ref[...] += part


def kernel(z, codebook):
    nz3 = jnp.sum(jnp.square(z), axis=2, keepdims=True)
    ne2 = jnp.sum(jnp.square(codebook), axis=1).reshape(1, _K)
    grid = (_B, _T // _TT)
    zq, sim, ids3, loss = pl.pallas_call(
        _vq_body,
        grid=grid,
        in_specs=[
            pl.BlockSpec((None, _TT, _D), lambda i, j: (i, j, 0)),
            pl.BlockSpec((_K, _D), lambda i, j: (0, 0)),
            pl.BlockSpec((None, _TT, 1), lambda i, j: (i, j, 0)),
            pl.BlockSpec((1, _K), lambda i, j: (0, 0)),
        ],
        out_specs=[
            pl.BlockSpec((None, _TT, _D), lambda i, j: (i, j, 0)),
            pl.BlockSpec((None, _TT, _K), lambda i, j: (i, j, 0)),
            pl.BlockSpec((None, _TT, 1), lambda i, j: (i, j, 0)),
            pl.BlockSpec((1, 1), lambda i, j: (0, 0)),
        ],
        out_shape=[
            jax.ShapeDtypeStruct((_B, _T, _D), jnp.float32),
            jax.ShapeDtypeStruct((_B, _T, _K), jnp.float32),
            jax.ShapeDtypeStruct((_B, _T, 1), jnp.int32),
            jax.ShapeDtypeStruct((1, 1), jnp.float32),
        ],
    )(z, codebook, nz3, ne2)
    ids = ids3.reshape(_B, _T)
    loss_vq = loss[0, 0] * (1.0 + _BETA) / (_B * _T)
    return zq, sim, ids, loss_vq


# speedup vs baseline: 2.2392x; 1.1885x over previous
import jax
import jax.numpy as jnp
from jax.experimental import pallas as pl

_B, _T, _D, _K = 16, 1024, 64, 1024
_TT = 512
_BETA = 0.25


def _vq_body(z_ref, cb_ref, nz_ref, ne_ref, zq_ref, sim_ref, ids_ref, loss_ref):
    i = pl.program_id(0)
    j = pl.program_id(1)
    z = z_ref[...]
    cb = cb_ref[...]
    nz = nz_ref[...]
    ne = ne_ref[...]
    dot = jax.lax.dot_general(z, cb, (((1,), (1,)), ((), ())),
                              preferred_element_type=jnp.float32)
    dist = -2.0 * dot + nz + ne
    m = jnp.min(dist, axis=1, keepdims=True)
    lane = jax.lax.broadcasted_iota(jnp.int32, (_TT, _K), 1)
    idx = jnp.min(jnp.where(dist == m, lane, _K), axis=1, keepdims=True)
    oh = (lane == idx).astype(jnp.float32)
    zq = jax.lax.dot_general(oh, cb, (((1,), (0,)), ((), ())),
                             preferred_element_type=jnp.float32)
    zq_ref[...] = zq
    sim_ref[...] = dot * jax.lax.rsqrt(nz) * jax.lax.rsqrt(ne)
    ids_ref[...] = idx
    norms = jnp.sqrt(jnp.maximum(m, 0.0))
    part = jnp.sum(norms, axis=0, keepdims=True)

    @pl.when(jnp.logical_and(i == 0, j == 0))
    def _():
        loss_ref[...] = jnp.zeros_like(loss_ref)

    loss_ref[...] += part


def kernel(z, codebook):
    nz3 = jnp.sum(jnp.square(z), axis=2, keepdims=True)
    ne2 = jnp.sum(jnp.square(codebook), axis=1).reshape(1, _K)
    grid = (_B, _T // _TT)
    zq, sim, ids3, loss = pl.pallas_call(
        _vq_body,
        grid=grid,
        in_specs=[
            pl.BlockSpec((None, _TT, _D), lambda i, j: (i, j, 0)),
            pl.BlockSpec((_K, _D), lambda i, j: (0, 0)),
            pl.BlockSpec((None, _TT, 1), lambda i, j: (i, j, 0)),
            pl.BlockSpec((1, _K), lambda i, j: (0, 0)),
        ],
        out_specs=[
            pl.BlockSpec((None, _TT, _D), lambda i, j: (i, j, 0)),
            pl.BlockSpec((None, _TT, _K), lambda i, j: (i, j, 0)),
            pl.BlockSpec((None, _TT, 1), lambda i, j: (i, j, 0)),
            pl.BlockSpec((1, 1), lambda i, j: (0, 0)),
        ],
        out_shape=[
            jax.ShapeDtypeStruct((_B, _T, _D), jnp.float32),
            jax.ShapeDtypeStruct((_B, _T, _K), jnp.float32),
            jax.ShapeDtypeStruct((_B, _T, 1), jnp.int32),
            jax.ShapeDtypeStruct((1, 1), jnp.float32),
        ],
    )(z, codebook, nz3, ne2)
    ids = ids3.reshape(_B, _T)
    loss_vq = loss[0, 0] * (1.0 + _BETA) / (_B * _T)
    return zq, sim, ids, loss_vq


# speedup vs baseline: 2.3220x; 1.0370x over previous
import jax
import jax.numpy as jnp
from jax.experimental import pallas as pl

_B, _T, _D, _K = 16, 1024, 64, 1024
_TT = 1024
_BETA = 0.25


def _vq_body(z_ref, cb_ref, nz_ref, ne_ref, zq_ref, sim_ref, ids_ref, loss_ref):
    i = pl.program_id(0)
    j = pl.program_id(1)
    z = z_ref[...]
    cb = cb_ref[...]
    nz = nz_ref[...]
    ne = ne_ref[...]
    dot = jax.lax.dot_general(z, cb, (((1,), (1,)), ((), ())),
                              preferred_element_type=jnp.float32)
    dist = -2.0 * dot + nz + ne
    m = jnp.min(dist, axis=1, keepdims=True)
    lane = jax.lax.broadcasted_iota(jnp.int32, (_TT, _K), 1)
    idx = jnp.min(jnp.where(dist == m, lane, _K), axis=1, keepdims=True)
    oh = (lane == idx).astype(jnp.float32)
    zq = jax.lax.dot_general(oh, cb, (((1,), (0,)), ((), ())),
                             preferred_element_type=jnp.float32)
    zq_ref[...] = zq
    sim_ref[...] = dot * jax.lax.rsqrt(nz) * jax.lax.rsqrt(ne)
    ids_ref[...] = idx
    norms = jnp.sqrt(jnp.maximum(m, 0.0))
    part = jnp.sum(norms, axis=0, keepdims=True)

    @pl.when(jnp.logical_and(i == 0, j == 0))
    def _():
        loss_ref[...] = jnp.zeros_like(loss_ref)

    loss_ref[...] += part


def kernel(z, codebook):
    nz3 = jnp.sum(jnp.square(z), axis=2, keepdims=True)
    ne2 = jnp.sum(jnp.square(codebook), axis=1).reshape(1, _K)
    grid = (_B, _T // _TT)
    zq, sim, ids3, loss = pl.pallas_call(
        _vq_body,
        grid=grid,
        in_specs=[
            pl.BlockSpec((None, _TT, _D), lambda i, j: (i, j, 0)),
            pl.BlockSpec((_K, _D), lambda i, j: (0, 0)),
            pl.BlockSpec((None, _TT, 1), lambda i, j: (i, j, 0)),
            pl.BlockSpec((1, _K), lambda i, j: (0, 0)),
        ],
        out_specs=[
            pl.BlockSpec((None, _TT, _D), lambda i, j: (i, j, 0)),
            pl.BlockSpec((None, _TT, _K), lambda i, j: (i, j, 0)),
            pl.BlockSpec((None, _TT, 1), lambda i, j: (i, j, 0)),
            pl.BlockSpec((1, 1), lambda i, j: (0, 0)),
        ],
        out_shape=[
            jax.ShapeDtypeStruct((_B, _T, _D), jnp.float32),
            jax.ShapeDtypeStruct((_B, _T, _K), jnp.float32),
            jax.ShapeDtypeStruct((_B, _T, 1), jnp.int32),
            jax.ShapeDtypeStruct((1, 1), jnp.float32),
        ],
    )(z, codebook, nz3, ne2)
    ids = ids3.reshape(_B, _T)
    loss_vq = loss[0, 0] * (1.0 + _BETA) / (_B * _T)
    return zq, sim, ids, loss_vq


# speedup vs baseline: 2.3486x; 1.0114x over previous
import jax
import jax.numpy as jnp
from jax.experimental import pallas as pl
from jax.experimental.pallas import tpu as pltpu

_B, _T, _D, _K = 16, 1024, 64, 1024
_BETA = 0.25


def _vq_body(z_ref, cb_ref, nz_ref, ne_ref, zq_ref, sim_ref, ids_ref, loss_ref):
    z = z_ref[...]
    cb = cb_ref[...]
    nz = nz_ref[...]
    ne = ne_ref[...]
    dot = jax.lax.dot_general(z, cb, (((1,), (1,)), ((), ())),
                              preferred_element_type=jnp.float32)
    dist = -2.0 * dot + nz + ne
    m = jnp.min(dist, axis=1, keepdims=True)
    lane = jax.lax.broadcasted_iota(jnp.int32, (_T, _K), 1)
    idx = jnp.min(jnp.where(dist == m, lane, _K), axis=1, keepdims=True)
    oh = (lane == idx).astype(jnp.float32)
    zq = jax.lax.dot_general(oh, cb, (((1,), (0,)), ((), ())),
                             preferred_element_type=jnp.float32)
    zq_ref[...] = zq
    sim_ref[...] = dot * jax.lax.rsqrt(nz) * jax.lax.rsqrt(ne)
    ids_ref[...] = idx
    norms = jnp.sqrt(jnp.maximum(m, 0.0))
    loss_ref[...] = jnp.sum(norms, axis=0, keepdims=True).reshape(1, 1)


def kernel(z, codebook):
    nz3 = jnp.sum(jnp.square(z), axis=2, keepdims=True)
    ne2 = jnp.sum(jnp.square(codebook), axis=1).reshape(1, _K)
    zq, sim, ids3, loss = pl.pallas_call(
        _vq_body,
        grid=(_B,),
        in_specs=[
            pl.BlockSpec((None, _T, _D), lambda i: (i, 0, 0)),
            pl.BlockSpec((_K, _D), lambda i: (0, 0)),
            pl.BlockSpec((None, _T, 1), lambda i: (i, 0, 0)),
            pl.BlockSpec((1, _K), lambda i: (0, 0)),
        ],
        out_specs=[
            pl.BlockSpec((None, _T, _D), lambda i: (i, 0, 0)),
            pl.BlockSpec((None, _T, _K), lambda i: (i, 0, 0)),
            pl.BlockSpec((None, _T, 1), lambda i: (i, 0, 0)),
            pl.BlockSpec((None, 1, 1), lambda i: (i, 0, 0)),
        ],
        out_shape=[
            jax.ShapeDtypeStruct((_B, _T, _D), jnp.float32),
            jax.ShapeDtypeStruct((_B, _T, _K), jnp.float32),
            jax.ShapeDtypeStruct((_B, _T, 1), jnp.int32),
            jax.ShapeDtypeStruct((_B, 1, 1), jnp.float32),
        ],
        compiler_params=pltpu.CompilerParams(
            dimension_semantics=("parallel",)),
    )(z, codebook, nz3, ne2)
    ids = ids3.reshape(_B, _T)
    loss_vq = jnp.sum(loss) * (1.0 + _BETA) / (_B * _T)
    return zq, sim, ids, loss_vq


# speedup vs baseline: 2.6231x; 1.1169x over previous
import jax
import jax.numpy as jnp
from jax.experimental import pallas as pl

_B, _T, _D, _K = 16, 1024, 64, 1024
_BETA = 0.25


def _vq_body(z_ref, cb_ref, nz_ref, ne_ref, zq_ref, sim_ref, ids_ref, loss_ref):
    i = pl.program_id(0)
    z = z_ref[...]
    cb = cb_ref[...]
    nz_row = nz_ref[pl.ds(i, 1), :]
    nz = jnp.transpose(nz_row, (1, 0))
    ne = ne_ref[...]
    dot = jax.lax.dot_general(z, cb, (((1,), (1,)), ((), ())),
                              preferred_element_type=jnp.float32)
    dist = -2.0 * dot + nz + ne
    m = jnp.min(dist, axis=1, keepdims=True)
    lane = jax.lax.broadcasted_iota(jnp.int32, (_T, _K), 1)
    idx = jnp.min(jnp.where(dist == m, lane, _K), axis=1, keepdims=True)
    oh = (lane == idx).astype(jnp.float32)
    zq = jax.lax.dot_general(oh, cb, (((1,), (0,)), ((), ())),
                             preferred_element_type=jnp.float32)
    zq_ref[...] = zq
    sim_ref[...] = dot * jax.lax.rsqrt(nz) * jax.lax.rsqrt(ne)
    ids_ref[pl.ds(i, 1), :] = jnp.transpose(idx, (1, 0))
    norms = jnp.sqrt(jnp.maximum(m, 0.0))
    loss_ref[...] = jnp.sum(norms, axis=0, keepdims=True).reshape(1, 1)


def kernel(z, codebook):
    nz2 = jnp.sum(jnp.square(z), axis=2)
    ne2 = jnp.sum(jnp.square(codebook), axis=1).reshape(1, _K)
    zq, sim, ids, loss = pl.pallas_call(
        _vq_body,
        grid=(_B,),
        in_specs=[
            pl.BlockSpec((None, _T, _D), lambda i: (i, 0, 0)),
            pl.BlockSpec((_K, _D), lambda i: (0, 0)),
            pl.BlockSpec((_B, _T), lambda i: (0, 0)),
            pl.BlockSpec((1, _K), lambda i: (0, 0)),
        ],
        out_specs=[
            pl.BlockSpec((None, _T, _D), lambda i: (i, 0, 0)),
            pl.BlockSpec((None, _T, _K), lambda i: (i, 0, 0)),
            pl.BlockSpec((_B, _T), lambda i: (0, 0)),
            pl.BlockSpec((None, 1, 1), lambda i: (i, 0, 0)),
        ],
        out_shape=[
            jax.ShapeDtypeStruct((_B, _T, _D), jnp.float32),
            jax.ShapeDtypeStruct((_B, _T, _K), jnp.float32),
            jax.ShapeDtypeStruct((_B, _T), jnp.int32),
            jax.ShapeDtypeStruct((_B, 1, 1), jnp.float32),
        ],
    )(z, codebook, nz2, ne2)
    loss_vq = jnp.sum(loss) * (1.0 + _BETA) / (_B * _T)
    return zq, sim, ids, loss_vq
